# baseline (speedup 1.0000x reference)
import functools

import jax
import jax.numpy as jnp
from jax import lax
from jax.experimental import pallas as pl
from jax.experimental.pallas import tpu as pltpu
from jax.experimental.pallas import tpu_sc as plsc

N = 10000
E = 320000
F_IN = 128
H = 16
C = 64

NC = 2
NS = 16
NW = NC * NS
CHUNK = 128
KCH = 80
RING = 8
DIST = 4
K0 = 96
K1 = 64
KMAX = max(K0, K1)
KCH_TOT = NS * K0 + NS * K1
E_PAD = KCH_TOT * CHUNK
ROWS_PER_TILE = 632
NPAD = NS * ROWS_PER_TILE

_BN = 2000
_GRID = N // _BN

_f32 = jnp.float32


def _tc1_body(x_ref, wl_ref, wr_ref, xl_ref, xr_ref):
    xb = x_ref[...]
    dn = (((1,), (1,)), ((), ()))
    xl_ref[...] = lax.dot_general(xb, wl_ref[...], dn, preferred_element_type=_f32)
    xr_ref[...] = lax.dot_general(xb, wr_ref[...], dn, preferred_element_type=_f32)


def _tc1(x, wl1, wr1):
    return pl.pallas_call(
        _tc1_body,
        grid=(_GRID,),
        in_specs=[
            pl.BlockSpec((_BN, F_IN), lambda i: (i, 0)),
            pl.BlockSpec((H, F_IN), lambda i: (0, 0)),
            pl.BlockSpec((H, F_IN), lambda i: (0, 0)),
        ],
        out_specs=[
            pl.BlockSpec((_BN, H), lambda i: (i, 0)),
            pl.BlockSpec((_BN, H), lambda i: (i, 0)),
        ],
        out_shape=[
            jax.ShapeDtypeStruct((N, H), _f32),
            jax.ShapeDtypeStruct((N, H), _f32),
        ],
    )(x, wl1, wr1)


def _mesh():
    return plsc.VectorSubcoreMesh(core_axis_name="c", subcore_axis_name="s")


def _make_segsum_body(with_deg):
    def body(*refs):
        if with_deg:
            (table, srcp, dstp, ones_hbm, zeros_hbm, acc_out, deg_out,
             src_v, dst_v, rows_v, ones_v, acc_s, deg_s, sem_g, sem_s) = refs
        else:
            (table, srcp, dstp, zeros_hbm, acc_out,
             src_v, dst_v, rows_v, acc_s, sem_g, sem_s) = refs
        cid = lax.axis_index("c")
        sid = lax.axis_index("s")
        base = sid * ROWS_PER_TILE

        off = jnp.where(cid == 0, sid * K0, NS * K0 + sid * K1)
        if with_deg:
            pltpu.sync_copy(ones_hbm, ones_v)
            pltpu.sync_copy(zeros_hbm, deg_s.at[pl.ds(base, ROWS_PER_TILE)])
        pltpu.sync_copy(zeros_hbm, acc_s.at[pl.ds(base, ROWS_PER_TILE)])
        plsc.subcore_barrier()

        def gather(j, b):
            pltpu.async_copy(table.at[src_v.at[j]], rows_v.at[b], sem_g.at[b])

        def gather_wait(j, b):
            pltpu.make_async_copy(table.at[src_v.at[j]], rows_v.at[b],
                                  sem_g.at[b]).wait()

        def scatter(j, b):
            pltpu.async_copy(rows_v.at[b], acc_s.at[dst_v.at[j]], sem_s.at[b],
                             add=True)
            if with_deg:
                pltpu.async_copy(ones_v, deg_s.at[dst_v.at[j]], sem_s.at[b],
                                 add=True)

        def scatter_wait(j, b):
            pltpu.make_async_copy(rows_v.at[b], acc_s.at[dst_v.at[j]],
                                  sem_s.at[b]).wait()
            if with_deg:
                pltpu.make_async_copy(ones_v, deg_s.at[dst_v.at[j]],
                                      sem_s.at[b]).wait()

        def pipeline(kch):
            pltpu.sync_copy(srcp.at[pl.ds(off, kch)],
                            src_v.at[pl.ds(0, kch)])
            pltpu.sync_copy(dstp.at[pl.ds(off, kch)],
                            dst_v.at[pl.ds(0, kch)])
            for j in range(DIST):
                gather(j, j)
            for j in range(DIST):
                gather_wait(j, j)
                scatter(j, j)
                gather(j + DIST, j + DIST)

            def group(g, carry):
                for b in range(RING):
                    j = g * RING + b + DIST
                    bb = (b + DIST) % RING
                    gather_wait(j, bb)
                    scatter(j, bb)
                    scatter_wait(j - DIST, b)
                    gather(j + DIST, b)
                return carry

            lax.fori_loop(0, (kch - 2 * DIST) // RING, group, 0)

            for t in range(DIST):
                j = kch - DIST + t
                gather_wait(j, j % RING)
                scatter(j, j % RING)
            for b in range(RING):
                scatter_wait(kch - RING + b, b)

        @pl.when(cid == 0)
        def _():
            pipeline(K0)

        @pl.when(cid == 1)
        def _():
            pipeline(K1)

        plsc.subcore_barrier()

        pltpu.sync_copy(acc_s.at[pl.ds(base, ROWS_PER_TILE)],
                        acc_out.at[cid].at[pl.ds(base, ROWS_PER_TILE)])
        if with_deg:
            pltpu.sync_copy(deg_s.at[pl.ds(base, ROWS_PER_TILE)],
                            deg_out.at[cid].at[pl.ds(base, ROWS_PER_TILE)])

    return body


_segsum_deg_kernel = _make_segsum_body(True)
_segsum_kernel = _make_segsum_body(False)


def _segsum_deg(*args):
    return pl.kernel(
        _segsum_deg_kernel,
        mesh=_mesh(),
        compiler_params=pltpu.CompilerParams(use_tc_tiling_on_sc=False),
        out_type=[
            jax.ShapeDtypeStruct((NC, NPAD, H), _f32),
            jax.ShapeDtypeStruct((NC, NPAD, H), _f32),
        ],
        scratch_types=[
            pltpu.VMEM((KMAX, CHUNK), jnp.int32),
            pltpu.VMEM((KMAX, CHUNK), jnp.int32),
            pltpu.VMEM((RING, CHUNK, H), _f32),
            pltpu.VMEM((CHUNK, H), _f32),
            pltpu.VMEM_SHARED((NPAD, H), _f32),
            pltpu.VMEM_SHARED((NPAD, H), _f32),
            pltpu.SemaphoreType.DMA((RING,)),
            pltpu.SemaphoreType.DMA((RING,)),
        ],
    )(*args)


def _segsum(*args):
    return pl.kernel(
        _segsum_kernel,
        mesh=_mesh(),
        compiler_params=pltpu.CompilerParams(use_tc_tiling_on_sc=False),
        out_type=jax.ShapeDtypeStruct((NC, NPAD, H), _f32),
        scratch_types=[
            pltpu.VMEM((KMAX, CHUNK), jnp.int32),
            pltpu.VMEM((KMAX, CHUNK), jnp.int32),
            pltpu.VMEM((RING, CHUNK, H), _f32),
            pltpu.VMEM_SHARED((NPAD, H), _f32),
            pltpu.SemaphoreType.DMA((RING,)),
            pltpu.SemaphoreType.DMA((RING,)),
        ],
    )(*args)


def _tc2_body(acc_ref, deg_ref, xr_ref, b_ref, wr2_ref, h_ref, hr_ref):
    agg = acc_ref[0] + acc_ref[1]
    deg = deg_ref[0, :, :1] + deg_ref[1, :, :1]
    pre = agg / jnp.maximum(deg, 1.0) + b_ref[...] + xr_ref[...]
    h = jnp.where(pre > 0, pre, jnp.exp(jnp.minimum(pre, 0.0)) - 1.0)
    h_ref[...] = h
    hr_ref[...] = lax.dot_general(h, wr2_ref[...], (((1,), (1,)), ((), ())),
                                  preferred_element_type=_f32)


def _tc2(acc1, degacc, xr, b1, wr2):
    return pl.pallas_call(
        _tc2_body,
        grid=(_GRID,),
        in_specs=[
            pl.BlockSpec((NC, _BN, H), lambda i: (0, i, 0)),
            pl.BlockSpec((NC, _BN, H), lambda i: (0, i, 0)),
            pl.BlockSpec((_BN, H), lambda i: (i, 0)),
            pl.BlockSpec((1, H), lambda i: (0, 0)),
            pl.BlockSpec((C, H), lambda i: (0, 0)),
        ],
        out_specs=[
            pl.BlockSpec((_BN, H), lambda i: (i, 0)),
            pl.BlockSpec((_BN, C), lambda i: (i, 0)),
        ],
        out_shape=[
            jax.ShapeDtypeStruct((N, H), _f32),
            jax.ShapeDtypeStruct((N, C), _f32),
        ],
    )(acc1, degacc, xr, b1, wr2)


def _tc3_body(acc_ref, deg_ref, hr_ref, b_ref, wl2_ref, out_ref):
    agg = acc_ref[0] + acc_ref[1]
    deg = deg_ref[0, :, :1] + deg_ref[1, :, :1]
    mean2 = agg / jnp.maximum(deg, 1.0)
    pre = lax.dot_general(mean2, wl2_ref[...], (((1,), (1,)), ((), ())),
                          preferred_element_type=_f32) + b_ref[...] + hr_ref[...]
    out_ref[...] = jnp.where(pre > 0, pre, jnp.exp(jnp.minimum(pre, 0.0)) - 1.0)


def _tc3(acc2, degacc, hr, b2, wl2):
    return pl.pallas_call(
        _tc3_body,
        grid=(_GRID,),
        in_specs=[
            pl.BlockSpec((NC, _BN, H), lambda i: (0, i, 0)),
            pl.BlockSpec((NC, _BN, H), lambda i: (0, i, 0)),
            pl.BlockSpec((_BN, C), lambda i: (i, 0)),
            pl.BlockSpec((1, C), lambda i: (0, 0)),
            pl.BlockSpec((C, H), lambda i: (0, 0)),
        ],
        out_specs=pl.BlockSpec((_BN, C), lambda i: (i, 0)),
        out_shape=jax.ShapeDtypeStruct((N, C), _f32),
    )(acc2, degacc, hr, b2, wl2)


def kernel(x, edge_index, W_l1, b_l1, W_r1, W_l2, b_l2, W_r2):
    src = edge_index[0]
    dst = edge_index[1]
    pad = E_PAD - E
    srcp = jnp.concatenate([src, jnp.zeros((pad,), jnp.int32)]).reshape(KCH_TOT, CHUNK)
    pad_dst = N + (jnp.arange(pad, dtype=jnp.int32) % (NPAD - N))
    dstp = jnp.concatenate([dst, pad_dst]).reshape(KCH_TOT, CHUNK)
    ones_hbm = jnp.ones((CHUNK, H), _f32)
    zeros_hbm = jnp.zeros((ROWS_PER_TILE, H), _f32)

    xl, xr = _tc1(x, W_l1, W_r1)
    acc1, degacc = _segsum_deg(xl, srcp, dstp, ones_hbm, zeros_hbm)
    h, hr = _tc2(acc1, degacc, xr, b_l1.reshape(1, H), W_r2)
    acc2 = _segsum(h, srcp, dstp, zeros_hbm)
    return _tc3(acc2, degacc, hr, b_l2.reshape(1, C), W_l2)


# speedup vs baseline: 1.0180x; 1.0180x over previous
import functools

import jax
import jax.numpy as jnp
from jax import lax
from jax.experimental import pallas as pl
from jax.experimental.pallas import tpu as pltpu
from jax.experimental.pallas import tpu_sc as plsc

N = 10000
E = 320000
F_IN = 128
H = 16
C = 64

NC = 2
NS = 16
NW = NC * NS
CHUNK = 128
KCH = 80
RING = 8
DIST = 4
K0 = 112
K1 = 48
KMAX = max(K0, K1)
KCH_TOT = NS * K0 + NS * K1
E_PAD = KCH_TOT * CHUNK
ROWS_PER_TILE = 632
NPAD = NS * ROWS_PER_TILE

_BN = 2000
_GRID = N // _BN

_f32 = jnp.float32


def _tc1_body(x_ref, wl_ref, wr_ref, xl_ref, xr_ref):
    xb = x_ref[...]
    dn = (((1,), (1,)), ((), ()))
    xl_ref[...] = lax.dot_general(xb, wl_ref[...], dn, preferred_element_type=_f32)
    xr_ref[...] = lax.dot_general(xb, wr_ref[...], dn, preferred_element_type=_f32)


def _tc1(x, wl1, wr1):
    return pl.pallas_call(
        _tc1_body,
        grid=(_GRID,),
        in_specs=[
            pl.BlockSpec((_BN, F_IN), lambda i: (i, 0)),
            pl.BlockSpec((H, F_IN), lambda i: (0, 0)),
            pl.BlockSpec((H, F_IN), lambda i: (0, 0)),
        ],
        out_specs=[
            pl.BlockSpec((_BN, H), lambda i: (i, 0)),
            pl.BlockSpec((_BN, H), lambda i: (i, 0)),
        ],
        out_shape=[
            jax.ShapeDtypeStruct((N, H), _f32),
            jax.ShapeDtypeStruct((N, H), _f32),
        ],
    )(x, wl1, wr1)


def _mesh():
    return plsc.VectorSubcoreMesh(core_axis_name="c", subcore_axis_name="s")


def _make_segsum_body(with_deg):
    def body(*refs):
        if with_deg:
            (table, srcp, dstp, ones_hbm, zeros_hbm, acc_out, deg_out,
             src_v, dst_v, rows_v, ones_v, acc_s, deg_s, sem_g, sem_s) = refs
        else:
            (table, srcp, dstp, zeros_hbm, acc_out,
             src_v, dst_v, rows_v, acc_s, sem_g, sem_s) = refs
        cid = lax.axis_index("c")
        sid = lax.axis_index("s")
        base = sid * ROWS_PER_TILE

        off = jnp.where(cid == 0, sid * K0, NS * K0 + sid * K1)
        if with_deg:
            pltpu.sync_copy(ones_hbm, ones_v)
            pltpu.sync_copy(zeros_hbm, deg_s.at[pl.ds(base, ROWS_PER_TILE)])
        pltpu.sync_copy(zeros_hbm, acc_s.at[pl.ds(base, ROWS_PER_TILE)])
        plsc.subcore_barrier()

        def gather(j, b):
            pltpu.async_copy(table.at[src_v.at[j]], rows_v.at[b], sem_g.at[b])

        def gather_wait(j, b):
            pltpu.make_async_copy(table.at[src_v.at[j]], rows_v.at[b],
                                  sem_g.at[b]).wait()

        def scatter(j, b):
            pltpu.async_copy(rows_v.at[b], acc_s.at[dst_v.at[j]], sem_s.at[b],
                             add=True)
            if with_deg:
                pltpu.async_copy(ones_v, deg_s.at[dst_v.at[j]], sem_s.at[b],
                                 add=True)

        def scatter_wait(j, b):
            pltpu.make_async_copy(rows_v.at[b], acc_s.at[dst_v.at[j]],
                                  sem_s.at[b]).wait()
            if with_deg:
                pltpu.make_async_copy(ones_v, deg_s.at[dst_v.at[j]],
                                      sem_s.at[b]).wait()

        def pipeline(kch):
            pltpu.sync_copy(srcp.at[pl.ds(off, kch)],
                            src_v.at[pl.ds(0, kch)])
            pltpu.sync_copy(dstp.at[pl.ds(off, kch)],
                            dst_v.at[pl.ds(0, kch)])
            for j in range(DIST):
                gather(j, j)
            for j in range(DIST):
                gather_wait(j, j)
                scatter(j, j)
                gather(j + DIST, j + DIST)

            def group(g, carry):
                for b in range(RING):
                    j = g * RING + b + DIST
                    bb = (b + DIST) % RING
                    gather_wait(j, bb)
                    scatter(j, bb)
                    scatter_wait(j - DIST, b)
                    gather(j + DIST, b)
                return carry

            lax.fori_loop(0, (kch - 2 * DIST) // RING, group, 0)

            for t in range(DIST):
                j = kch - DIST + t
                gather_wait(j, j % RING)
                scatter(j, j % RING)
            for b in range(RING):
                scatter_wait(kch - RING + b, b)

        @pl.when(cid == 0)
        def _():
            pipeline(K0)

        @pl.when(cid == 1)
        def _():
            pipeline(K1)

        plsc.subcore_barrier()

        pltpu.sync_copy(acc_s.at[pl.ds(base, ROWS_PER_TILE)],
                        acc_out.at[cid].at[pl.ds(base, ROWS_PER_TILE)])
        if with_deg:
            pltpu.sync_copy(deg_s.at[pl.ds(base, ROWS_PER_TILE)],
                            deg_out.at[cid].at[pl.ds(base, ROWS_PER_TILE)])

    return body


_segsum_deg_kernel = _make_segsum_body(True)
_segsum_kernel = _make_segsum_body(False)


def _segsum_deg(*args):
    return pl.kernel(
        _segsum_deg_kernel,
        mesh=_mesh(),
        compiler_params=pltpu.CompilerParams(use_tc_tiling_on_sc=False),
        out_type=[
            jax.ShapeDtypeStruct((NC, NPAD, H), _f32),
            jax.ShapeDtypeStruct((NC, NPAD, H), _f32),
        ],
        scratch_types=[
            pltpu.VMEM((KMAX, CHUNK), jnp.int32),
            pltpu.VMEM((KMAX, CHUNK), jnp.int32),
            pltpu.VMEM((RING, CHUNK, H), _f32),
            pltpu.VMEM((CHUNK, H), _f32),
            pltpu.VMEM_SHARED((NPAD, H), _f32),
            pltpu.VMEM_SHARED((NPAD, H), _f32),
            pltpu.SemaphoreType.DMA((RING,)),
            pltpu.SemaphoreType.DMA((RING,)),
        ],
    )(*args)


def _segsum(*args):
    return pl.kernel(
        _segsum_kernel,
        mesh=_mesh(),
        compiler_params=pltpu.CompilerParams(use_tc_tiling_on_sc=False),
        out_type=jax.ShapeDtypeStruct((NC, NPAD, H), _f32),
        scratch_types=[
            pltpu.VMEM((KMAX, CHUNK), jnp.int32),
            pltpu.VMEM((KMAX, CHUNK), jnp.int32),
            pltpu.VMEM((RING, CHUNK, H), _f32),
            pltpu.VMEM_SHARED((NPAD, H), _f32),
            pltpu.SemaphoreType.DMA((RING,)),
            pltpu.SemaphoreType.DMA((RING,)),
        ],
    )(*args)


def _tc2_body(acc_ref, deg_ref, xr_ref, b_ref, wr2_ref, h_ref, hr_ref):
    agg = acc_ref[0] + acc_ref[1]
    deg = deg_ref[0, :, :1] + deg_ref[1, :, :1]
    pre = agg / jnp.maximum(deg, 1.0) + b_ref[...] + xr_ref[...]
    h = jnp.where(pre > 0, pre, jnp.exp(jnp.minimum(pre, 0.0)) - 1.0)
    h_ref[...] = h
    hr_ref[...] = lax.dot_general(h, wr2_ref[...], (((1,), (1,)), ((), ())),
                                  preferred_element_type=_f32)


def _tc2(acc1, degacc, xr, b1, wr2):
    return pl.pallas_call(
        _tc2_body,
        grid=(_GRID,),
        in_specs=[
            pl.BlockSpec((NC, _BN, H), lambda i: (0, i, 0)),
            pl.BlockSpec((NC, _BN, H), lambda i: (0, i, 0)),
            pl.BlockSpec((_BN, H), lambda i: (i, 0)),
            pl.BlockSpec((1, H), lambda i: (0, 0)),
            pl.BlockSpec((C, H), lambda i: (0, 0)),
        ],
        out_specs=[
            pl.BlockSpec((_BN, H), lambda i: (i, 0)),
            pl.BlockSpec((_BN, C), lambda i: (i, 0)),
        ],
        out_shape=[
            jax.ShapeDtypeStruct((N, H), _f32),
            jax.ShapeDtypeStruct((N, C), _f32),
        ],
    )(acc1, degacc, xr, b1, wr2)


def _tc3_body(acc_ref, deg_ref, hr_ref, b_ref, wl2_ref, out_ref):
    agg = acc_ref[0] + acc_ref[1]
    deg = deg_ref[0, :, :1] + deg_ref[1, :, :1]
    mean2 = agg / jnp.maximum(deg, 1.0)
    pre = lax.dot_general(mean2, wl2_ref[...], (((1,), (1,)), ((), ())),
                          preferred_element_type=_f32) + b_ref[...] + hr_ref[...]
    out_ref[...] = jnp.where(pre > 0, pre, jnp.exp(jnp.minimum(pre, 0.0)) - 1.0)


def _tc3(acc2, degacc, hr, b2, wl2):
    return pl.pallas_call(
        _tc3_body,
        grid=(_GRID,),
        in_specs=[
            pl.BlockSpec((NC, _BN, H), lambda i: (0, i, 0)),
            pl.BlockSpec((NC, _BN, H), lambda i: (0, i, 0)),
            pl.BlockSpec((_BN, C), lambda i: (i, 0)),
            pl.BlockSpec((1, C), lambda i: (0, 0)),
            pl.BlockSpec((C, H), lambda i: (0, 0)),
        ],
        out_specs=pl.BlockSpec((_BN, C), lambda i: (i, 0)),
        out_shape=jax.ShapeDtypeStruct((N, C), _f32),
    )(acc2, degacc, hr, b2, wl2)


def kernel(x, edge_index, W_l1, b_l1, W_r1, W_l2, b_l2, W_r2):
    src = edge_index[0]
    dst = edge_index[1]
    pad = E_PAD - E
    srcp = jnp.concatenate([src, jnp.zeros((pad,), jnp.int32)]).reshape(KCH_TOT, CHUNK)
    pad_dst = N + (jnp.arange(pad, dtype=jnp.int32) % (NPAD - N))
    dstp = jnp.concatenate([dst, pad_dst]).reshape(KCH_TOT, CHUNK)
    ones_hbm = jnp.ones((CHUNK, H), _f32)
    zeros_hbm = jnp.zeros((ROWS_PER_TILE, H), _f32)

    xl, xr = _tc1(x, W_l1, W_r1)
    acc1, degacc = _segsum_deg(xl, srcp, dstp, ones_hbm, zeros_hbm)
    h, hr = _tc2(acc1, degacc, xr, b_l1.reshape(1, H), W_r2)
    acc2 = _segsum(h, srcp, dstp, zeros_hbm)
    return _tc3(acc2, degacc, hr, b_l2.reshape(1, C), W_l2)


# speedup vs baseline: 1.0217x; 1.0036x over previous
import functools

import jax
import jax.numpy as jnp
from jax import lax
from jax.experimental import pallas as pl
from jax.experimental.pallas import tpu as pltpu
from jax.experimental.pallas import tpu_sc as plsc

N = 10000
E = 320000
F_IN = 128
H = 16
C = 64

NC = 2
NS = 16
NW = NC * NS
CHUNK = 128
KCH = 80
RING = 8
DIST = 4
K0 = 120
K1 = 40
KMAX = max(K0, K1)
KCH_TOT = NS * K0 + NS * K1
E_PAD = KCH_TOT * CHUNK
ROWS_PER_TILE = 632
NPAD = NS * ROWS_PER_TILE

_BN = 2000
_GRID = N // _BN

_f32 = jnp.float32


def _tc1_body(x_ref, wl_ref, wr_ref, xl_ref, xr_ref):
    xb = x_ref[...]
    dn = (((1,), (1,)), ((), ()))
    xl_ref[...] = lax.dot_general(xb, wl_ref[...], dn, preferred_element_type=_f32)
    xr_ref[...] = lax.dot_general(xb, wr_ref[...], dn, preferred_element_type=_f32)


def _tc1(x, wl1, wr1):
    return pl.pallas_call(
        _tc1_body,
        grid=(_GRID,),
        in_specs=[
            pl.BlockSpec((_BN, F_IN), lambda i: (i, 0)),
            pl.BlockSpec((H, F_IN), lambda i: (0, 0)),
            pl.BlockSpec((H, F_IN), lambda i: (0, 0)),
        ],
        out_specs=[
            pl.BlockSpec((_BN, H), lambda i: (i, 0)),
            pl.BlockSpec((_BN, H), lambda i: (i, 0)),
        ],
        out_shape=[
            jax.ShapeDtypeStruct((N, H), _f32),
            jax.ShapeDtypeStruct((N, H), _f32),
        ],
    )(x, wl1, wr1)


def _mesh():
    return plsc.VectorSubcoreMesh(core_axis_name="c", subcore_axis_name="s")


def _make_segsum_body(with_deg):
    def body(*refs):
        if with_deg:
            (table, srcp, dstp, ones_hbm, zeros_hbm, acc_out, deg_out,
             src_v, dst_v, rows_v, ones_v, acc_s, deg_s, sem_g, sem_s) = refs
        else:
            (table, srcp, dstp, zeros_hbm, acc_out,
             src_v, dst_v, rows_v, acc_s, sem_g, sem_s) = refs
        cid = lax.axis_index("c")
        sid = lax.axis_index("s")
        base = sid * ROWS_PER_TILE

        off = jnp.where(cid == 0, sid * K0, NS * K0 + sid * K1)
        if with_deg:
            pltpu.sync_copy(ones_hbm, ones_v)
            pltpu.sync_copy(zeros_hbm, deg_s.at[pl.ds(base, ROWS_PER_TILE)])
        pltpu.sync_copy(zeros_hbm, acc_s.at[pl.ds(base, ROWS_PER_TILE)])
        plsc.subcore_barrier()

        def gather(j, b):
            pltpu.async_copy(table.at[src_v.at[j]], rows_v.at[b], sem_g.at[b])

        def gather_wait(j, b):
            pltpu.make_async_copy(table.at[src_v.at[j]], rows_v.at[b],
                                  sem_g.at[b]).wait()

        def scatter(j, b):
            pltpu.async_copy(rows_v.at[b], acc_s.at[dst_v.at[j]], sem_s.at[b],
                             add=True)
            if with_deg:
                pltpu.async_copy(ones_v, deg_s.at[dst_v.at[j]], sem_s.at[b],
                                 add=True)

        def scatter_wait(j, b):
            pltpu.make_async_copy(rows_v.at[b], acc_s.at[dst_v.at[j]],
                                  sem_s.at[b]).wait()
            if with_deg:
                pltpu.make_async_copy(ones_v, deg_s.at[dst_v.at[j]],
                                      sem_s.at[b]).wait()

        def pipeline(kch):
            pltpu.sync_copy(srcp.at[pl.ds(off, kch)],
                            src_v.at[pl.ds(0, kch)])
            pltpu.sync_copy(dstp.at[pl.ds(off, kch)],
                            dst_v.at[pl.ds(0, kch)])
            for j in range(DIST):
                gather(j, j)
            for j in range(DIST):
                gather_wait(j, j)
                scatter(j, j)
                gather(j + DIST, j + DIST)

            def group(g, carry):
                for b in range(RING):
                    j = g * RING + b + DIST
                    bb = (b + DIST) % RING
                    gather_wait(j, bb)
                    scatter(j, bb)
                    scatter_wait(j - DIST, b)
                    gather(j + DIST, b)
                return carry

            lax.fori_loop(0, (kch - 2 * DIST) // RING, group, 0)

            for t in range(DIST):
                j = kch - DIST + t
                gather_wait(j, j % RING)
                scatter(j, j % RING)
            for b in range(RING):
                scatter_wait(kch - RING + b, b)

        @pl.when(cid == 0)
        def _():
            pipeline(K0)

        @pl.when(cid == 1)
        def _():
            pipeline(K1)

        plsc.subcore_barrier()

        pltpu.sync_copy(acc_s.at[pl.ds(base, ROWS_PER_TILE)],
                        acc_out.at[cid].at[pl.ds(base, ROWS_PER_TILE)])
        if with_deg:
            pltpu.sync_copy(deg_s.at[pl.ds(base, ROWS_PER_TILE)],
                            deg_out.at[cid].at[pl.ds(base, ROWS_PER_TILE)])

    return body


_segsum_deg_kernel = _make_segsum_body(True)
_segsum_kernel = _make_segsum_body(False)


def _segsum_deg(*args):
    return pl.kernel(
        _segsum_deg_kernel,
        mesh=_mesh(),
        compiler_params=pltpu.CompilerParams(use_tc_tiling_on_sc=False),
        out_type=[
            jax.ShapeDtypeStruct((NC, NPAD, H), _f32),
            jax.ShapeDtypeStruct((NC, NPAD, H), _f32),
        ],
        scratch_types=[
            pltpu.VMEM((KMAX, CHUNK), jnp.int32),
            pltpu.VMEM((KMAX, CHUNK), jnp.int32),
            pltpu.VMEM((RING, CHUNK, H), _f32),
            pltpu.VMEM((CHUNK, H), _f32),
            pltpu.VMEM_SHARED((NPAD, H), _f32),
            pltpu.VMEM_SHARED((NPAD, H), _f32),
            pltpu.SemaphoreType.DMA((RING,)),
            pltpu.SemaphoreType.DMA((RING,)),
        ],
    )(*args)


def _segsum(*args):
    return pl.kernel(
        _segsum_kernel,
        mesh=_mesh(),
        compiler_params=pltpu.CompilerParams(use_tc_tiling_on_sc=False),
        out_type=jax.ShapeDtypeStruct((NC, NPAD, H), _f32),
        scratch_types=[
            pltpu.VMEM((KMAX, CHUNK), jnp.int32),
            pltpu.VMEM((KMAX, CHUNK), jnp.int32),
            pltpu.VMEM((RING, CHUNK, H), _f32),
            pltpu.VMEM_SHARED((NPAD, H), _f32),
            pltpu.SemaphoreType.DMA((RING,)),
            pltpu.SemaphoreType.DMA((RING,)),
        ],
    )(*args)


def _tc2_body(acc_ref, deg_ref, xr_ref, b_ref, wr2_ref, h_ref, hr_ref):
    agg = acc_ref[0] + acc_ref[1]
    deg = deg_ref[0, :, :1] + deg_ref[1, :, :1]
    pre = agg / jnp.maximum(deg, 1.0) + b_ref[...] + xr_ref[...]
    h = jnp.where(pre > 0, pre, jnp.exp(jnp.minimum(pre, 0.0)) - 1.0)
    h_ref[...] = h
    hr_ref[...] = lax.dot_general(h, wr2_ref[...], (((1,), (1,)), ((), ())),
                                  preferred_element_type=_f32)


def _tc2(acc1, degacc, xr, b1, wr2):
    return pl.pallas_call(
        _tc2_body,
        grid=(_GRID,),
        in_specs=[
            pl.BlockSpec((NC, _BN, H), lambda i: (0, i, 0)),
            pl.BlockSpec((NC, _BN, H), lambda i: (0, i, 0)),
            pl.BlockSpec((_BN, H), lambda i: (i, 0)),
            pl.BlockSpec((1, H), lambda i: (0, 0)),
            pl.BlockSpec((C, H), lambda i: (0, 0)),
        ],
        out_specs=[
            pl.BlockSpec((_BN, H), lambda i: (i, 0)),
            pl.BlockSpec((_BN, C), lambda i: (i, 0)),
        ],
        out_shape=[
            jax.ShapeDtypeStruct((N, H), _f32),
            jax.ShapeDtypeStruct((N, C), _f32),
        ],
    )(acc1, degacc, xr, b1, wr2)


def _tc3_body(acc_ref, deg_ref, hr_ref, b_ref, wl2_ref, out_ref):
    agg = acc_ref[0] + acc_ref[1]
    deg = deg_ref[0, :, :1] + deg_ref[1, :, :1]
    mean2 = agg / jnp.maximum(deg, 1.0)
    pre = lax.dot_general(mean2, wl2_ref[...], (((1,), (1,)), ((), ())),
                          preferred_element_type=_f32) + b_ref[...] + hr_ref[...]
    out_ref[...] = jnp.where(pre > 0, pre, jnp.exp(jnp.minimum(pre, 0.0)) - 1.0)


def _tc3(acc2, degacc, hr, b2, wl2):
    return pl.pallas_call(
        _tc3_body,
        grid=(_GRID,),
        in_specs=[
            pl.BlockSpec((NC, _BN, H), lambda i: (0, i, 0)),
            pl.BlockSpec((NC, _BN, H), lambda i: (0, i, 0)),
            pl.BlockSpec((_BN, C), lambda i: (i, 0)),
            pl.BlockSpec((1, C), lambda i: (0, 0)),
            pl.BlockSpec((C, H), lambda i: (0, 0)),
        ],
        out_specs=pl.BlockSpec((_BN, C), lambda i: (i, 0)),
        out_shape=jax.ShapeDtypeStruct((N, C), _f32),
    )(acc2, degacc, hr, b2, wl2)


def kernel(x, edge_index, W_l1, b_l1, W_r1, W_l2, b_l2, W_r2):
    src = edge_index[0]
    dst = edge_index[1]
    pad = E_PAD - E
    srcp = jnp.concatenate([src, jnp.zeros((pad,), jnp.int32)]).reshape(KCH_TOT, CHUNK)
    pad_dst = N + (jnp.arange(pad, dtype=jnp.int32) % (NPAD - N))
    dstp = jnp.concatenate([dst, pad_dst]).reshape(KCH_TOT, CHUNK)
    ones_hbm = jnp.ones((CHUNK, H), _f32)
    zeros_hbm = jnp.zeros((ROWS_PER_TILE, H), _f32)

    xl, xr = _tc1(x, W_l1, W_r1)
    acc1, degacc = _segsum_deg(xl, srcp, dstp, ones_hbm, zeros_hbm)
    h, hr = _tc2(acc1, degacc, xr, b_l1.reshape(1, H), W_r2)
    acc2 = _segsum(h, srcp, dstp, zeros_hbm)
    return _tc3(acc2, degacc, hr, b_l2.reshape(1, C), W_l2)


# speedup vs baseline: 1.0235x; 1.0017x over previous
import functools

import jax
import jax.numpy as jnp
from jax import lax
from jax.experimental import pallas as pl
from jax.experimental.pallas import tpu as pltpu
from jax.experimental.pallas import tpu_sc as plsc

N = 10000
E = 320000
F_IN = 128
H = 16
C = 64

NC = 2
NS = 16
NW = NC * NS
CHUNK = 128
KCH = 80
RING = 8
DIST = 4
K0 = 128
K1 = 32
KMAX = max(K0, K1)
KCH_TOT = NS * K0 + NS * K1
E_PAD = KCH_TOT * CHUNK
ROWS_PER_TILE = 632
NPAD = NS * ROWS_PER_TILE

_BN = 2000
_GRID = N // _BN

_f32 = jnp.float32


def _tc1_body(x_ref, wl_ref, wr_ref, xl_ref, xr_ref):
    xb = x_ref[...]
    dn = (((1,), (1,)), ((), ()))
    xl_ref[...] = lax.dot_general(xb, wl_ref[...], dn, preferred_element_type=_f32)
    xr_ref[...] = lax.dot_general(xb, wr_ref[...], dn, preferred_element_type=_f32)


def _tc1(x, wl1, wr1):
    return pl.pallas_call(
        _tc1_body,
        grid=(_GRID,),
        in_specs=[
            pl.BlockSpec((_BN, F_IN), lambda i: (i, 0)),
            pl.BlockSpec((H, F_IN), lambda i: (0, 0)),
            pl.BlockSpec((H, F_IN), lambda i: (0, 0)),
        ],
        out_specs=[
            pl.BlockSpec((_BN, H), lambda i: (i, 0)),
            pl.BlockSpec((_BN, H), lambda i: (i, 0)),
        ],
        out_shape=[
            jax.ShapeDtypeStruct((N, H), _f32),
            jax.ShapeDtypeStruct((N, H), _f32),
        ],
    )(x, wl1, wr1)


def _mesh():
    return plsc.VectorSubcoreMesh(core_axis_name="c", subcore_axis_name="s")


def _make_segsum_body(with_deg):
    def body(*refs):
        if with_deg:
            (table, srcp, dstp, ones_hbm, zeros_hbm, acc_out, deg_out,
             src_v, dst_v, rows_v, ones_v, acc_s, deg_s, sem_g, sem_s) = refs
        else:
            (table, srcp, dstp, zeros_hbm, acc_out,
             src_v, dst_v, rows_v, acc_s, sem_g, sem_s) = refs
        cid = lax.axis_index("c")
        sid = lax.axis_index("s")
        base = sid * ROWS_PER_TILE

        off = jnp.where(cid == 0, sid * K0, NS * K0 + sid * K1)
        if with_deg:
            pltpu.sync_copy(ones_hbm, ones_v)
            pltpu.sync_copy(zeros_hbm, deg_s.at[pl.ds(base, ROWS_PER_TILE)])
        pltpu.sync_copy(zeros_hbm, acc_s.at[pl.ds(base, ROWS_PER_TILE)])
        plsc.subcore_barrier()

        def gather(j, b):
            pltpu.async_copy(table.at[src_v.at[j]], rows_v.at[b], sem_g.at[b])

        def gather_wait(j, b):
            pltpu.make_async_copy(table.at[src_v.at[j]], rows_v.at[b],
                                  sem_g.at[b]).wait()

        def scatter(j, b):
            pltpu.async_copy(rows_v.at[b], acc_s.at[dst_v.at[j]], sem_s.at[b],
                             add=True)
            if with_deg:
                pltpu.async_copy(ones_v, deg_s.at[dst_v.at[j]], sem_s.at[b],
                                 add=True)

        def scatter_wait(j, b):
            pltpu.make_async_copy(rows_v.at[b], acc_s.at[dst_v.at[j]],
                                  sem_s.at[b]).wait()
            if with_deg:
                pltpu.make_async_copy(ones_v, deg_s.at[dst_v.at[j]],
                                      sem_s.at[b]).wait()

        def pipeline(kch):
            pltpu.sync_copy(srcp.at[pl.ds(off, kch)],
                            src_v.at[pl.ds(0, kch)])
            pltpu.sync_copy(dstp.at[pl.ds(off, kch)],
                            dst_v.at[pl.ds(0, kch)])
            for j in range(DIST):
                gather(j, j)
            for j in range(DIST):
                gather_wait(j, j)
                scatter(j, j)
                gather(j + DIST, j + DIST)

            def group(g, carry):
                for b in range(RING):
                    j = g * RING + b + DIST
                    bb = (b + DIST) % RING
                    gather_wait(j, bb)
                    scatter(j, bb)
                    scatter_wait(j - DIST, b)
                    gather(j + DIST, b)
                return carry

            lax.fori_loop(0, (kch - 2 * DIST) // RING, group, 0)

            for t in range(DIST):
                j = kch - DIST + t
                gather_wait(j, j % RING)
                scatter(j, j % RING)
            for b in range(RING):
                scatter_wait(kch - RING + b, b)

        @pl.when(cid == 0)
        def _():
            pipeline(K0)

        @pl.when(cid == 1)
        def _():
            pipeline(K1)

        plsc.subcore_barrier()

        pltpu.sync_copy(acc_s.at[pl.ds(base, ROWS_PER_TILE)],
                        acc_out.at[cid].at[pl.ds(base, ROWS_PER_TILE)])
        if with_deg:
            pltpu.sync_copy(deg_s.at[pl.ds(base, ROWS_PER_TILE)],
                            deg_out.at[cid].at[pl.ds(base, ROWS_PER_TILE)])

    return body


_segsum_deg_kernel = _make_segsum_body(True)
_segsum_kernel = _make_segsum_body(False)


def _segsum_deg(*args):
    return pl.kernel(
        _segsum_deg_kernel,
        mesh=_mesh(),
        compiler_params=pltpu.CompilerParams(use_tc_tiling_on_sc=False),
        out_type=[
            jax.ShapeDtypeStruct((NC, NPAD, H), _f32),
            jax.ShapeDtypeStruct((NC, NPAD, H), _f32),
        ],
        scratch_types=[
            pltpu.VMEM((KMAX, CHUNK), jnp.int32),
            pltpu.VMEM((KMAX, CHUNK), jnp.int32),
            pltpu.VMEM((RING, CHUNK, H), _f32),
            pltpu.VMEM((CHUNK, H), _f32),
            pltpu.VMEM_SHARED((NPAD, H), _f32),
            pltpu.VMEM_SHARED((NPAD, H), _f32),
            pltpu.SemaphoreType.DMA((RING,)),
            pltpu.SemaphoreType.DMA((RING,)),
        ],
    )(*args)


def _segsum(*args):
    return pl.kernel(
        _segsum_kernel,
        mesh=_mesh(),
        compiler_params=pltpu.CompilerParams(use_tc_tiling_on_sc=False),
        out_type=jax.ShapeDtypeStruct((NC, NPAD, H), _f32),
        scratch_types=[
            pltpu.VMEM((KMAX, CHUNK), jnp.int32),
            pltpu.VMEM((KMAX, CHUNK), jnp.int32),
            pltpu.VMEM((RING, CHUNK, H), _f32),
            pltpu.VMEM_SHARED((NPAD, H), _f32),
            pltpu.SemaphoreType.DMA((RING,)),
            pltpu.SemaphoreType.DMA((RING,)),
        ],
    )(*args)


def _tc2_body(acc_ref, deg_ref, xr_ref, b_ref, wr2_ref, h_ref, hr_ref):
    agg = acc_ref[0] + acc_ref[1]
    deg = deg_ref[0, :, :1] + deg_ref[1, :, :1]
    pre = agg / jnp.maximum(deg, 1.0) + b_ref[...] + xr_ref[...]
    h = jnp.where(pre > 0, pre, jnp.exp(jnp.minimum(pre, 0.0)) - 1.0)
    h_ref[...] = h
    hr_ref[...] = lax.dot_general(h, wr2_ref[...], (((1,), (1,)), ((), ())),
                                  preferred_element_type=_f32)


def _tc2(acc1, degacc, xr, b1, wr2):
    return pl.pallas_call(
        _tc2_body,
        grid=(_GRID,),
        in_specs=[
            pl.BlockSpec((NC, _BN, H), lambda i: (0, i, 0)),
            pl.BlockSpec((NC, _BN, H), lambda i: (0, i, 0)),
            pl.BlockSpec((_BN, H), lambda i: (i, 0)),
            pl.BlockSpec((1, H), lambda i: (0, 0)),
            pl.BlockSpec((C, H), lambda i: (0, 0)),
        ],
        out_specs=[
            pl.BlockSpec((_BN, H), lambda i: (i, 0)),
            pl.BlockSpec((_BN, C), lambda i: (i, 0)),
        ],
        out_shape=[
            jax.ShapeDtypeStruct((N, H), _f32),
            jax.ShapeDtypeStruct((N, C), _f32),
        ],
    )(acc1, degacc, xr, b1, wr2)


def _tc3_body(acc_ref, deg_ref, hr_ref, b_ref, wl2_ref, out_ref):
    agg = acc_ref[0] + acc_ref[1]
    deg = deg_ref[0, :, :1] + deg_ref[1, :, :1]
    mean2 = agg / jnp.maximum(deg, 1.0)
    pre = lax.dot_general(mean2, wl2_ref[...], (((1,), (1,)), ((), ())),
                          preferred_element_type=_f32) + b_ref[...] + hr_ref[...]
    out_ref[...] = jnp.where(pre > 0, pre, jnp.exp(jnp.minimum(pre, 0.0)) - 1.0)


def _tc3(acc2, degacc, hr, b2, wl2):
    return pl.pallas_call(
        _tc3_body,
        grid=(_GRID,),
        in_specs=[
            pl.BlockSpec((NC, _BN, H), lambda i: (0, i, 0)),
            pl.BlockSpec((NC, _BN, H), lambda i: (0, i, 0)),
            pl.BlockSpec((_BN, C), lambda i: (i, 0)),
            pl.BlockSpec((1, C), lambda i: (0, 0)),
            pl.BlockSpec((C, H), lambda i: (0, 0)),
        ],
        out_specs=pl.BlockSpec((_BN, C), lambda i: (i, 0)),
        out_shape=jax.ShapeDtypeStruct((N, C), _f32),
    )(acc2, degacc, hr, b2, wl2)


def kernel(x, edge_index, W_l1, b_l1, W_r1, W_l2, b_l2, W_r2):
    src = edge_index[0]
    dst = edge_index[1]
    pad = E_PAD - E
    srcp = jnp.concatenate([src, jnp.zeros((pad,), jnp.int32)]).reshape(KCH_TOT, CHUNK)
    pad_dst = N + (jnp.arange(pad, dtype=jnp.int32) % (NPAD - N))
    dstp = jnp.concatenate([dst, pad_dst]).reshape(KCH_TOT, CHUNK)
    ones_hbm = jnp.ones((CHUNK, H), _f32)
    zeros_hbm = jnp.zeros((ROWS_PER_TILE, H), _f32)

    xl, xr = _tc1(x, W_l1, W_r1)
    acc1, degacc = _segsum_deg(xl, srcp, dstp, ones_hbm, zeros_hbm)
    h, hr = _tc2(acc1, degacc, xr, b_l1.reshape(1, H), W_r2)
    acc2 = _segsum(h, srcp, dstp, zeros_hbm)
    return _tc3(acc2, degacc, hr, b_l2.reshape(1, C), W_l2)


# speedup vs baseline: 1.0383x; 1.0145x over previous
import functools

import jax
import jax.numpy as jnp
from jax import lax
from jax.experimental import pallas as pl
from jax.experimental.pallas import tpu as pltpu
from jax.experimental.pallas import tpu_sc as plsc

N = 10000
E = 320000
F_IN = 128
H = 16
C = 64

NC = 2
NS = 16
NW = NC * NS
CHUNK = 128
KCH = 80
RING = 8
DIST = 4
K0 = 136
K1 = 24
KMAX = max(K0, K1)
KCH_TOT = NS * K0 + NS * K1
E_PAD = KCH_TOT * CHUNK
ROWS_PER_TILE = 632
NPAD = NS * ROWS_PER_TILE

_BN = 2000
_GRID = N // _BN

_f32 = jnp.float32


def _tc1_body(x_ref, wl_ref, wr_ref, xl_ref, xr_ref):
    xb = x_ref[...]
    dn = (((1,), (1,)), ((), ()))
    xl_ref[...] = lax.dot_general(xb, wl_ref[...], dn, preferred_element_type=_f32)
    xr_ref[...] = lax.dot_general(xb, wr_ref[...], dn, preferred_element_type=_f32)


def _tc1(x, wl1, wr1):
    return pl.pallas_call(
        _tc1_body,
        grid=(_GRID,),
        in_specs=[
            pl.BlockSpec((_BN, F_IN), lambda i: (i, 0)),
            pl.BlockSpec((H, F_IN), lambda i: (0, 0)),
            pl.BlockSpec((H, F_IN), lambda i: (0, 0)),
        ],
        out_specs=[
            pl.BlockSpec((_BN, H), lambda i: (i, 0)),
            pl.BlockSpec((_BN, H), lambda i: (i, 0)),
        ],
        out_shape=[
            jax.ShapeDtypeStruct((N, H), _f32),
            jax.ShapeDtypeStruct((N, H), _f32),
        ],
    )(x, wl1, wr1)


def _mesh():
    return plsc.VectorSubcoreMesh(core_axis_name="c", subcore_axis_name="s")


def _make_segsum_body(with_deg):
    def body(*refs):
        if with_deg:
            (table, srcp, dstp, ones_hbm, zeros_hbm, acc_out, deg_out,
             src_v, dst_v, rows_v, ones_v, acc_s, deg_s, sem_g, sem_s) = refs
        else:
            (table, srcp, dstp, zeros_hbm, acc_out,
             src_v, dst_v, rows_v, acc_s, sem_g, sem_s) = refs
        cid = lax.axis_index("c")
        sid = lax.axis_index("s")
        base = sid * ROWS_PER_TILE

        off = jnp.where(cid == 0, sid * K0, NS * K0 + sid * K1)
        if with_deg:
            pltpu.sync_copy(ones_hbm, ones_v)
            pltpu.sync_copy(zeros_hbm, deg_s.at[pl.ds(base, ROWS_PER_TILE)])
        pltpu.sync_copy(zeros_hbm, acc_s.at[pl.ds(base, ROWS_PER_TILE)])
        plsc.subcore_barrier()

        def gather(j, b):
            pltpu.async_copy(table.at[src_v.at[j]], rows_v.at[b], sem_g.at[b])

        def gather_wait(j, b):
            pltpu.make_async_copy(table.at[src_v.at[j]], rows_v.at[b],
                                  sem_g.at[b]).wait()

        def scatter(j, b):
            pltpu.async_copy(rows_v.at[b], acc_s.at[dst_v.at[j]], sem_s.at[b],
                             add=True)
            if with_deg:
                pltpu.async_copy(ones_v, deg_s.at[dst_v.at[j]], sem_s.at[b],
                                 add=True)

        def scatter_wait(j, b):
            pltpu.make_async_copy(rows_v.at[b], acc_s.at[dst_v.at[j]],
                                  sem_s.at[b]).wait()
            if with_deg:
                pltpu.make_async_copy(ones_v, deg_s.at[dst_v.at[j]],
                                      sem_s.at[b]).wait()

        def pipeline(kch):
            pltpu.sync_copy(srcp.at[pl.ds(off, kch)],
                            src_v.at[pl.ds(0, kch)])
            pltpu.sync_copy(dstp.at[pl.ds(off, kch)],
                            dst_v.at[pl.ds(0, kch)])
            for j in range(DIST):
                gather(j, j)
            for j in range(DIST):
                gather_wait(j, j)
                scatter(j, j)
                gather(j + DIST, j + DIST)

            def group(g, carry):
                for b in range(RING):
                    j = g * RING + b + DIST
                    bb = (b + DIST) % RING
                    gather_wait(j, bb)
                    scatter(j, bb)
                    scatter_wait(j - DIST, b)
                    gather(j + DIST, b)
                return carry

            lax.fori_loop(0, (kch - 2 * DIST) // RING, group, 0)

            for t in range(DIST):
                j = kch - DIST + t
                gather_wait(j, j % RING)
                scatter(j, j % RING)
            for b in range(RING):
                scatter_wait(kch - RING + b, b)

        @pl.when(cid == 0)
        def _():
            pipeline(K0)

        @pl.when(cid == 1)
        def _():
            pipeline(K1)

        plsc.subcore_barrier()

        pltpu.sync_copy(acc_s.at[pl.ds(base, ROWS_PER_TILE)],
                        acc_out.at[cid].at[pl.ds(base, ROWS_PER_TILE)])
        if with_deg:
            pltpu.sync_copy(deg_s.at[pl.ds(base, ROWS_PER_TILE)],
                            deg_out.at[cid].at[pl.ds(base, ROWS_PER_TILE)])

    return body


_segsum_deg_kernel = _make_segsum_body(True)
_segsum_kernel = _make_segsum_body(False)


def _segsum_deg(*args):
    return pl.kernel(
        _segsum_deg_kernel,
        mesh=_mesh(),
        compiler_params=pltpu.CompilerParams(use_tc_tiling_on_sc=False),
        out_type=[
            jax.ShapeDtypeStruct((NC, NPAD, H), _f32),
            jax.ShapeDtypeStruct((NC, NPAD, H), _f32),
        ],
        scratch_types=[
            pltpu.VMEM((KMAX, CHUNK), jnp.int32),
            pltpu.VMEM((KMAX, CHUNK), jnp.int32),
            pltpu.VMEM((RING, CHUNK, H), _f32),
            pltpu.VMEM((CHUNK, H), _f32),
            pltpu.VMEM_SHARED((NPAD, H), _f32),
            pltpu.VMEM_SHARED((NPAD, H), _f32),
            pltpu.SemaphoreType.DMA((RING,)),
            pltpu.SemaphoreType.DMA((RING,)),
        ],
    )(*args)


def _segsum(*args):
    return pl.kernel(
        _segsum_kernel,
        mesh=_mesh(),
        compiler_params=pltpu.CompilerParams(use_tc_tiling_on_sc=False),
        out_type=jax.ShapeDtypeStruct((NC, NPAD, H), _f32),
        scratch_types=[
            pltpu.VMEM((KMAX, CHUNK), jnp.int32),
            pltpu.VMEM((KMAX, CHUNK), jnp.int32),
            pltpu.VMEM((RING, CHUNK, H), _f32),
            pltpu.VMEM_SHARED((NPAD, H), _f32),
            pltpu.SemaphoreType.DMA((RING,)),
            pltpu.SemaphoreType.DMA((RING,)),
        ],
    )(*args)


def _tc2_body(acc_ref, deg_ref, xr_ref, b_ref, wr2_ref, h_ref, hr_ref):
    agg = acc_ref[0] + acc_ref[1]
    deg = deg_ref[0, :, :1] + deg_ref[1, :, :1]
    pre = agg / jnp.maximum(deg, 1.0) + b_ref[...] + xr_ref[...]
    h = jnp.where(pre > 0, pre, jnp.exp(jnp.minimum(pre, 0.0)) - 1.0)
    h_ref[...] = h
    hr_ref[...] = lax.dot_general(h, wr2_ref[...], (((1,), (1,)), ((), ())),
                                  preferred_element_type=_f32)


def _tc2(acc1, degacc, xr, b1, wr2):
    return pl.pallas_call(
        _tc2_body,
        grid=(_GRID,),
        in_specs=[
            pl.BlockSpec((NC, _BN, H), lambda i: (0, i, 0)),
            pl.BlockSpec((NC, _BN, H), lambda i: (0, i, 0)),
            pl.BlockSpec((_BN, H), lambda i: (i, 0)),
            pl.BlockSpec((1, H), lambda i: (0, 0)),
            pl.BlockSpec((C, H), lambda i: (0, 0)),
        ],
        out_specs=[
            pl.BlockSpec((_BN, H), lambda i: (i, 0)),
            pl.BlockSpec((_BN, C), lambda i: (i, 0)),
        ],
        out_shape=[
            jax.ShapeDtypeStruct((N, H), _f32),
            jax.ShapeDtypeStruct((N, C), _f32),
        ],
    )(acc1, degacc, xr, b1, wr2)


def _tc3_body(acc_ref, deg_ref, hr_ref, b_ref, wl2_ref, out_ref):
    agg = acc_ref[0] + acc_ref[1]
    deg = deg_ref[0, :, :1] + deg_ref[1, :, :1]
    mean2 = agg / jnp.maximum(deg, 1.0)
    pre = lax.dot_general(mean2, wl2_ref[...], (((1,), (1,)), ((), ())),
                          preferred_element_type=_f32) + b_ref[...] + hr_ref[...]
    out_ref[...] = jnp.where(pre > 0, pre, jnp.exp(jnp.minimum(pre, 0.0)) - 1.0)


def _tc3(acc2, degacc, hr, b2, wl2):
    return pl.pallas_call(
        _tc3_body,
        grid=(_GRID,),
        in_specs=[
            pl.BlockSpec((NC, _BN, H), lambda i: (0, i, 0)),
            pl.BlockSpec((NC, _BN, H), lambda i: (0, i, 0)),
            pl.BlockSpec((_BN, C), lambda i: (i, 0)),
            pl.BlockSpec((1, C), lambda i: (0, 0)),
            pl.BlockSpec((C, H), lambda i: (0, 0)),
        ],
        out_specs=pl.BlockSpec((_BN, C), lambda i: (i, 0)),
        out_shape=jax.ShapeDtypeStruct((N, C), _f32),
    )(acc2, degacc, hr, b2, wl2)


def kernel(x, edge_index, W_l1, b_l1, W_r1, W_l2, b_l2, W_r2):
    src = edge_index[0]
    dst = edge_index[1]
    pad = E_PAD - E
    srcp = jnp.concatenate([src, jnp.zeros((pad,), jnp.int32)]).reshape(KCH_TOT, CHUNK)
    pad_dst = N + (jnp.arange(pad, dtype=jnp.int32) % (NPAD - N))
    dstp = jnp.concatenate([dst, pad_dst]).reshape(KCH_TOT, CHUNK)
    ones_hbm = jnp.ones((CHUNK, H), _f32)
    zeros_hbm = jnp.zeros((ROWS_PER_TILE, H), _f32)

    xl, xr = _tc1(x, W_l1, W_r1)
    acc1, degacc = _segsum_deg(xl, srcp, dstp, ones_hbm, zeros_hbm)
    h, hr = _tc2(acc1, degacc, xr, b_l1.reshape(1, H), W_r2)
    acc2 = _segsum(h, srcp, dstp, zeros_hbm)
    return _tc3(acc2, degacc, hr, b_l2.reshape(1, C), W_l2)


# speedup vs baseline: 1.0659x; 1.0266x over previous
import functools

import jax
import jax.numpy as jnp
from jax import lax
from jax.experimental import pallas as pl
from jax.experimental.pallas import tpu as pltpu
from jax.experimental.pallas import tpu_sc as plsc

N = 10000
E = 320000
F_IN = 128
H = 16
C = 64

NC = 2
NS = 16
NW = NC * NS
CHUNK = 128
KCH = 80
RING = 8
DIST = 4
K0 = 144
K1 = 16
KMAX = max(K0, K1)
KCH_TOT = NS * K0 + NS * K1
E_PAD = KCH_TOT * CHUNK
ROWS_PER_TILE = 632
NPAD = NS * ROWS_PER_TILE

_BN = 2000
_GRID = N // _BN

_f32 = jnp.float32


def _tc1_body(x_ref, wl_ref, wr_ref, xl_ref, xr_ref):
    xb = x_ref[...]
    dn = (((1,), (1,)), ((), ()))
    xl_ref[...] = lax.dot_general(xb, wl_ref[...], dn, preferred_element_type=_f32)
    xr_ref[...] = lax.dot_general(xb, wr_ref[...], dn, preferred_element_type=_f32)


def _tc1(x, wl1, wr1):
    return pl.pallas_call(
        _tc1_body,
        grid=(_GRID,),
        in_specs=[
            pl.BlockSpec((_BN, F_IN), lambda i: (i, 0)),
            pl.BlockSpec((H, F_IN), lambda i: (0, 0)),
            pl.BlockSpec((H, F_IN), lambda i: (0, 0)),
        ],
        out_specs=[
            pl.BlockSpec((_BN, H), lambda i: (i, 0)),
            pl.BlockSpec((_BN, H), lambda i: (i, 0)),
        ],
        out_shape=[
            jax.ShapeDtypeStruct((N, H), _f32),
            jax.ShapeDtypeStruct((N, H), _f32),
        ],
    )(x, wl1, wr1)


def _mesh():
    return plsc.VectorSubcoreMesh(core_axis_name="c", subcore_axis_name="s")


def _make_segsum_body(with_deg):
    def body(*refs):
        if with_deg:
            (table, srcp, dstp, ones_hbm, zeros_hbm, acc_out, deg_out,
             src_v, dst_v, rows_v, ones_v, acc_s, deg_s, sem_g, sem_s) = refs
        else:
            (table, srcp, dstp, zeros_hbm, acc_out,
             src_v, dst_v, rows_v, acc_s, sem_g, sem_s) = refs
        cid = lax.axis_index("c")
        sid = lax.axis_index("s")
        base = sid * ROWS_PER_TILE

        off = jnp.where(cid == 0, sid * K0, NS * K0 + sid * K1)
        if with_deg:
            pltpu.sync_copy(ones_hbm, ones_v)
            pltpu.sync_copy(zeros_hbm, deg_s.at[pl.ds(base, ROWS_PER_TILE)])
        pltpu.sync_copy(zeros_hbm, acc_s.at[pl.ds(base, ROWS_PER_TILE)])
        plsc.subcore_barrier()

        def gather(j, b):
            pltpu.async_copy(table.at[src_v.at[j]], rows_v.at[b], sem_g.at[b])

        def gather_wait(j, b):
            pltpu.make_async_copy(table.at[src_v.at[j]], rows_v.at[b],
                                  sem_g.at[b]).wait()

        def scatter(j, b):
            pltpu.async_copy(rows_v.at[b], acc_s.at[dst_v.at[j]], sem_s.at[b],
                             add=True)
            if with_deg:
                pltpu.async_copy(ones_v, deg_s.at[dst_v.at[j]], sem_s.at[b],
                                 add=True)

        def scatter_wait(j, b):
            pltpu.make_async_copy(rows_v.at[b], acc_s.at[dst_v.at[j]],
                                  sem_s.at[b]).wait()
            if with_deg:
                pltpu.make_async_copy(ones_v, deg_s.at[dst_v.at[j]],
                                      sem_s.at[b]).wait()

        def pipeline(kch):
            pltpu.sync_copy(srcp.at[pl.ds(off, kch)],
                            src_v.at[pl.ds(0, kch)])
            pltpu.sync_copy(dstp.at[pl.ds(off, kch)],
                            dst_v.at[pl.ds(0, kch)])
            for j in range(DIST):
                gather(j, j)
            for j in range(DIST):
                gather_wait(j, j)
                scatter(j, j)
                gather(j + DIST, j + DIST)

            def group(g, carry):
                for b in range(RING):
                    j = g * RING + b + DIST
                    bb = (b + DIST) % RING
                    gather_wait(j, bb)
                    scatter(j, bb)
                    scatter_wait(j - DIST, b)
                    gather(j + DIST, b)
                return carry

            lax.fori_loop(0, (kch - 2 * DIST) // RING, group, 0)

            for t in range(DIST):
                j = kch - DIST + t
                gather_wait(j, j % RING)
                scatter(j, j % RING)
            for b in range(RING):
                scatter_wait(kch - RING + b, b)

        @pl.when(cid == 0)
        def _():
            pipeline(K0)

        @pl.when(cid == 1)
        def _():
            pipeline(K1)

        plsc.subcore_barrier()

        pltpu.sync_copy(acc_s.at[pl.ds(base, ROWS_PER_TILE)],
                        acc_out.at[cid].at[pl.ds(base, ROWS_PER_TILE)])
        if with_deg:
            pltpu.sync_copy(deg_s.at[pl.ds(base, ROWS_PER_TILE)],
                            deg_out.at[cid].at[pl.ds(base, ROWS_PER_TILE)])

    return body


_segsum_deg_kernel = _make_segsum_body(True)
_segsum_kernel = _make_segsum_body(False)


def _segsum_deg(*args):
    return pl.kernel(
        _segsum_deg_kernel,
        mesh=_mesh(),
        compiler_params=pltpu.CompilerParams(use_tc_tiling_on_sc=False),
        out_type=[
            jax.ShapeDtypeStruct((NC, NPAD, H), _f32),
            jax.ShapeDtypeStruct((NC, NPAD, H), _f32),
        ],
        scratch_types=[
            pltpu.VMEM((KMAX, CHUNK), jnp.int32),
            pltpu.VMEM((KMAX, CHUNK), jnp.int32),
            pltpu.VMEM((RING, CHUNK, H), _f32),
            pltpu.VMEM((CHUNK, H), _f32),
            pltpu.VMEM_SHARED((NPAD, H), _f32),
            pltpu.VMEM_SHARED((NPAD, H), _f32),
            pltpu.SemaphoreType.DMA((RING,)),
            pltpu.SemaphoreType.DMA((RING,)),
        ],
    )(*args)


def _segsum(*args):
    return pl.kernel(
        _segsum_kernel,
        mesh=_mesh(),
        compiler_params=pltpu.CompilerParams(use_tc_tiling_on_sc=False),
        out_type=jax.ShapeDtypeStruct((NC, NPAD, H), _f32),
        scratch_types=[
            pltpu.VMEM((KMAX, CHUNK), jnp.int32),
            pltpu.VMEM((KMAX, CHUNK), jnp.int32),
            pltpu.VMEM((RING, CHUNK, H), _f32),
            pltpu.VMEM_SHARED((NPAD, H), _f32),
            pltpu.SemaphoreType.DMA((RING,)),
            pltpu.SemaphoreType.DMA((RING,)),
        ],
    )(*args)


def _tc2_body(acc_ref, deg_ref, xr_ref, b_ref, wr2_ref, h_ref, hr_ref):
    agg = acc_ref[0] + acc_ref[1]
    deg = deg_ref[0, :, :1] + deg_ref[1, :, :1]
    pre = agg / jnp.maximum(deg, 1.0) + b_ref[...] + xr_ref[...]
    h = jnp.where(pre > 0, pre, jnp.exp(jnp.minimum(pre, 0.0)) - 1.0)
    h_ref[...] = h
    hr_ref[...] = lax.dot_general(h, wr2_ref[...], (((1,), (1,)), ((), ())),
                                  preferred_element_type=_f32)


def _tc2(acc1, degacc, xr, b1, wr2):
    return pl.pallas_call(
        _tc2_body,
        grid=(_GRID,),
        in_specs=[
            pl.BlockSpec((NC, _BN, H), lambda i: (0, i, 0)),
            pl.BlockSpec((NC, _BN, H), lambda i: (0, i, 0)),
            pl.BlockSpec((_BN, H), lambda i: (i, 0)),
            pl.BlockSpec((1, H), lambda i: (0, 0)),
            pl.BlockSpec((C, H), lambda i: (0, 0)),
        ],
        out_specs=[
            pl.BlockSpec((_BN, H), lambda i: (i, 0)),
            pl.BlockSpec((_BN, C), lambda i: (i, 0)),
        ],
        out_shape=[
            jax.ShapeDtypeStruct((N, H), _f32),
            jax.ShapeDtypeStruct((N, C), _f32),
        ],
    )(acc1, degacc, xr, b1, wr2)


def _tc3_body(acc_ref, deg_ref, hr_ref, b_ref, wl2_ref, out_ref):
    agg = acc_ref[0] + acc_ref[1]
    deg = deg_ref[0, :, :1] + deg_ref[1, :, :1]
    mean2 = agg / jnp.maximum(deg, 1.0)
    pre = lax.dot_general(mean2, wl2_ref[...], (((1,), (1,)), ((), ())),
                          preferred_element_type=_f32) + b_ref[...] + hr_ref[...]
    out_ref[...] = jnp.where(pre > 0, pre, jnp.exp(jnp.minimum(pre, 0.0)) - 1.0)


def _tc3(acc2, degacc, hr, b2, wl2):
    return pl.pallas_call(
        _tc3_body,
        grid=(_GRID,),
        in_specs=[
            pl.BlockSpec((NC, _BN, H), lambda i: (0, i, 0)),
            pl.BlockSpec((NC, _BN, H), lambda i: (0, i, 0)),
            pl.BlockSpec((_BN, C), lambda i: (i, 0)),
            pl.BlockSpec((1, C), lambda i: (0, 0)),
            pl.BlockSpec((C, H), lambda i: (0, 0)),
        ],
        out_specs=pl.BlockSpec((_BN, C), lambda i: (i, 0)),
        out_shape=jax.ShapeDtypeStruct((N, C), _f32),
    )(acc2, degacc, hr, b2, wl2)


def kernel(x, edge_index, W_l1, b_l1, W_r1, W_l2, b_l2, W_r2):
    src = edge_index[0]
    dst = edge_index[1]
    pad = E_PAD - E
    srcp = jnp.concatenate([src, jnp.zeros((pad,), jnp.int32)]).reshape(KCH_TOT, CHUNK)
    pad_dst = N + (jnp.arange(pad, dtype=jnp.int32) % (NPAD - N))
    dstp = jnp.concatenate([dst, pad_dst]).reshape(KCH_TOT, CHUNK)
    ones_hbm = jnp.ones((CHUNK, H), _f32)
    zeros_hbm = jnp.zeros((ROWS_PER_TILE, H), _f32)

    xl, xr = _tc1(x, W_l1, W_r1)
    acc1, degacc = _segsum_deg(xl, srcp, dstp, ones_hbm, zeros_hbm)
    h, hr = _tc2(acc1, degacc, xr, b_l1.reshape(1, H), W_r2)
    acc2 = _segsum(h, srcp, dstp, zeros_hbm)
    return _tc3(acc2, degacc, hr, b_l2.reshape(1, C), W_l2)
